# baseline (speedup 1.0000x reference)
import jax
import jax.numpy as jnp
from jax import lax
from jax.experimental import pallas as pl
from jax.experimental.pallas import tpu as pltpu
from jax.experimental.pallas import tpu_sc as plsc

BATCH = 4
SEQ_LEN = 8192
D_MODEL = 768
N_ROWS = BATCH * SEQ_LEN

NUM_CORES = 2
NUM_SUBCORES = 16
NUM_WORKERS = NUM_CORES * NUM_SUBCORES
ROWS_PER_WORKER = N_ROWS // NUM_WORKERS
CHUNK = 32
N_CHUNKS = ROWS_PER_WORKER // CHUNK
NX = 3
NR = 2
PERIOD = 6


def _pos_enc_body(x_hbm, idx_hbm, table_hbm, out_hbm, idx_v,
                  bufx0, bufx1, bufx2, bufr0, bufr1, semx, semr, semo):
    bufx = [bufx0, bufx1, bufx2]
    bufr = [bufr0, bufr1]
    wid = lax.axis_index("s") * NUM_CORES + lax.axis_index("c")
    base = wid * ROWS_PER_WORKER
    pltpu.sync_copy(idx_hbm.at[pl.ds(base, ROWS_PER_WORKER)], idx_v)

    def issue_x(c, xs):
        pltpu.async_copy(
            x_hbm.at[pl.ds(base + c * CHUNK, CHUNK)], bufx[xs], semx.at[xs]
        )

    def issue_gather(c, rs):
        pltpu.async_copy(
            table_hbm.at[idx_v.at[pl.ds(c * CHUNK, CHUNK)]],
            bufr[rs],
            semr.at[rs],
        )

    def issue_store(c, xs):
        pltpu.async_copy(
            bufx[xs], out_hbm.at[pl.ds(base + c * CHUNK, CHUNK)], semo.at[xs]
        )

    def wait_x(xs):
        pltpu.make_async_copy(
            x_hbm.at[pl.ds(0, CHUNK)], bufx[xs], semx.at[xs]
        ).wait()

    def wait_r(rs):
        pltpu.make_async_copy(
            x_hbm.at[pl.ds(0, CHUNK)], bufr[rs], semr.at[rs]
        ).wait()

    def wait_o(xs):
        pltpu.make_async_copy(
            bufx[xs], out_hbm.at[pl.ds(0, CHUNK)], semo.at[xs]
        ).wait()

    def alu(xs, rs):
        bx = bufx[xs]
        br = bufr[rs]

        def row_body(k, rcarry):
            r = k * 2
            for rr in range(2):
                for j in range(D_MODEL // 16):
                    s = pl.ds(j * 16, 16)
                    bx[r + rr, s] = bx[r + rr, s] + br[r + rr, s]
            return rcarry

        lax.fori_loop(0, CHUNK // 2, row_body, 0)

    def chunk_body(c, i, drain_store, more_x, more_gather):
        xs = i % NX
        rs = i % NR
        if more_x:
            nxs = (i + 1) % NX
            if drain_store:
                wait_o(nxs)
            issue_x(c + 1, nxs)
        wait_r(rs)
        wait_x(xs)
        alu(xs, rs)
        issue_store(c, xs)
        if more_gather:
            issue_gather(c + 2, rs)

    issue_x(0, 0)
    issue_gather(0, 0)
    issue_gather(1, 1)

    for i in range(PERIOD):
        chunk_body(i, i, i >= 2, True, True)

    def period_body(g, carry):
        c0 = g * PERIOD
        for i in range(PERIOD):
            chunk_body(c0 + i, i, True, True, True)
        return carry

    lax.fori_loop(1, (N_CHUNKS - 2) // PERIOD, period_body, 0)

    chunk_body(30, 0, True, True, False)
    chunk_body(31, 1, False, False, False)

    wait_o(2)
    wait_o(0)
    wait_o(1)


@jax.jit
def kernel(x, position_ids, pos_table):
    x2 = x.reshape(N_ROWS, D_MODEL)
    idx = position_ids.astype(jnp.int32).reshape(N_ROWS)

    mesh = plsc.VectorSubcoreMesh(
        core_axis_name="c",
        subcore_axis_name="s",
        num_cores=NUM_CORES,
        num_subcores=NUM_SUBCORES,
    )
    out = pl.kernel(
        _pos_enc_body,
        out_type=jax.ShapeDtypeStruct((N_ROWS, D_MODEL), jnp.float32),
        mesh=mesh,
        scratch_types=[
            pltpu.VMEM((ROWS_PER_WORKER,), jnp.int32),
            pltpu.VMEM((CHUNK, D_MODEL), jnp.float32),
            pltpu.VMEM((CHUNK, D_MODEL), jnp.float32),
            pltpu.VMEM((CHUNK, D_MODEL), jnp.float32),
            pltpu.VMEM((CHUNK, D_MODEL), jnp.float32),
            pltpu.VMEM((CHUNK, D_MODEL), jnp.float32),
            pltpu.SemaphoreType.DMA((NX,)),
            pltpu.SemaphoreType.DMA((NR,)),
            pltpu.SemaphoreType.DMA((NX,)),
        ],
    )(x2, idx, pos_table)
    return out.reshape(BATCH, SEQ_LEN, D_MODEL)


# speedup vs baseline: 1.0186x; 1.0186x over previous
import jax
import jax.numpy as jnp
from jax import lax
from jax.experimental import pallas as pl
from jax.experimental.pallas import tpu as pltpu
from jax.experimental.pallas import tpu_sc as plsc

BATCH = 4
SEQ_LEN = 8192
D_MODEL = 768
N_ROWS = BATCH * SEQ_LEN

NUM_CORES = 2
NUM_SUBCORES = 16
NUM_WORKERS = NUM_CORES * NUM_SUBCORES
ROWS_PER_WORKER = N_ROWS // NUM_WORKERS
CHUNK = 32
N_CHUNKS = ROWS_PER_WORKER // CHUNK
NX = 3
NR = 2
PERIOD = 6


def _pos_enc_body(x_hbm, idx_hbm, table_hbm, out_hbm, idx_v,
                  bufx0, bufx1, bufx2, bufr0, bufr1, semx, semr, semo):
    bufx = [bufx0, bufx1, bufx2]
    bufr = [bufr0, bufr1]
    wid = lax.axis_index("s") * NUM_CORES + lax.axis_index("c")
    base = wid * ROWS_PER_WORKER
    pltpu.sync_copy(idx_hbm.at[pl.ds(base, ROWS_PER_WORKER)], idx_v)

    def issue_x(c, xs):
        pltpu.async_copy(
            x_hbm.at[pl.ds(base + c * CHUNK, CHUNK)], bufx[xs], semx.at[xs]
        )

    def issue_gather(c, rs):
        pltpu.async_copy(
            table_hbm.at[idx_v.at[pl.ds(c * CHUNK, CHUNK)]],
            bufr[rs],
            semr.at[rs],
        )

    def issue_store(c, xs):
        pltpu.async_copy(
            bufx[xs], out_hbm.at[pl.ds(base + c * CHUNK, CHUNK)], semo.at[xs]
        )

    def wait_x(xs):
        pltpu.make_async_copy(
            x_hbm.at[pl.ds(0, CHUNK)], bufx[xs], semx.at[xs]
        ).wait()

    def wait_r(rs):
        pltpu.make_async_copy(
            x_hbm.at[pl.ds(0, CHUNK)], bufr[rs], semr.at[rs]
        ).wait()

    def wait_o(xs):
        pltpu.make_async_copy(
            bufx[xs], out_hbm.at[pl.ds(0, CHUNK)], semo.at[xs]
        ).wait()

    def alu(xs, rs):
        bx = bufx[xs]
        br = bufr[rs]

        def row_body(r, rcarry):
            for j in range(D_MODEL // 16):
                s = pl.ds(j * 16, 16)
                bx[r, s] = bx[r, s] + br[r, s]
            return rcarry

        lax.fori_loop(0, CHUNK, row_body, 0)

    def chunk_body(c, i, drain_store, more_x, more_gather):
        xs = i % NX
        rs = i % NR
        if more_x:
            nxs = (i + 1) % NX
            if drain_store:
                wait_o(nxs)
            issue_x(c + 1, nxs)
        wait_r(rs)
        wait_x(xs)
        alu(xs, rs)
        issue_store(c, xs)
        if more_gather:
            issue_gather(c + 2, rs)

    issue_x(0, 0)
    issue_gather(0, 0)
    issue_gather(1, 1)

    for i in range(PERIOD):
        chunk_body(i, i, i >= 2, True, True)

    def period_body(g, carry):
        c0 = g * PERIOD
        for i in range(PERIOD):
            chunk_body(c0 + i, i, True, True, True)
        return carry

    lax.fori_loop(1, (N_CHUNKS - 2) // PERIOD, period_body, 0)

    chunk_body(30, 0, True, True, False)
    chunk_body(31, 1, False, False, False)

    wait_o(2)
    wait_o(0)
    wait_o(1)


@jax.jit
def kernel(x, position_ids, pos_table):
    x2 = x.reshape(N_ROWS, D_MODEL)
    idx = position_ids.astype(jnp.int32).reshape(N_ROWS)

    mesh = plsc.VectorSubcoreMesh(
        core_axis_name="c",
        subcore_axis_name="s",
        num_cores=NUM_CORES,
        num_subcores=NUM_SUBCORES,
    )
    out = pl.kernel(
        _pos_enc_body,
        out_type=jax.ShapeDtypeStruct((N_ROWS, D_MODEL), jnp.float32),
        mesh=mesh,
        scratch_types=[
            pltpu.VMEM((ROWS_PER_WORKER,), jnp.int32),
            pltpu.VMEM((CHUNK, D_MODEL), jnp.float32),
            pltpu.VMEM((CHUNK, D_MODEL), jnp.float32),
            pltpu.VMEM((CHUNK, D_MODEL), jnp.float32),
            pltpu.VMEM((CHUNK, D_MODEL), jnp.float32),
            pltpu.VMEM((CHUNK, D_MODEL), jnp.float32),
            pltpu.SemaphoreType.DMA((NX,)),
            pltpu.SemaphoreType.DMA((NR,)),
            pltpu.SemaphoreType.DMA((NX,)),
        ],
    )(x2, idx, pos_table)
    return out.reshape(BATCH, SEQ_LEN, D_MODEL)
